# baseline (speedup 1.0000x reference)
import functools

import jax
import jax.numpy as jnp
from jax import lax
from jax.experimental import pallas as pl
from jax.experimental.pallas import tpu as pltpu
from jax.experimental.pallas import tpu_sc as plsc

MAX_LEN = 8192


def _sc_broadcast_rows(b, s, d):
    info = plsc.get_sparse_core_info()
    nw = info.num_cores * info.num_subcores
    assert s % nw == 0, (s, nw)
    rows_per_w = s // nw

    mesh = plsc.VectorSubcoreMesh(core_axis_name="c", subcore_axis_name="s")

    @functools.partial(
        pl.kernel,
        mesh=mesh,
        out_type=jax.ShapeDtypeStruct((b, s, d), jnp.float32),
        scratch_types=[
            pltpu.SemaphoreType.DMA,
        ],
    )
    def prog(emb_hbm, out_hbm, sem):
        wid = lax.axis_index("s") * info.num_cores + lax.axis_index("c")
        base = wid * rows_per_w
        src = emb_hbm.at[pl.ds(base, rows_per_w)]
        handles = [
            pltpu.async_copy(src, out_hbm.at[bb, pl.ds(base, rows_per_w)], sem)
            for bb in range(b)
        ]
        for h in handles:
            h.wait()

    return prog


def kernel(x, emb_weight):
    b, s, _ = x.shape
    d = emb_weight.shape[1]
    if s >= MAX_LEN:
        raise NotImplementedError("s >= MAX_SEQ_LEN not exercised by this problem")
    prog = _sc_broadcast_rows(b, s, d)
    return prog(emb_weight)


# speedup vs baseline: 39.9227x; 39.9227x over previous
import functools

import jax
import jax.numpy as jnp
from jax import lax
from jax.experimental import pallas as pl
from jax.experimental.pallas import tpu as pltpu
from jax.experimental.pallas import tpu_sc as plsc

MAX_LEN = 8192


def _sc_broadcast_rows(b, s, d):
    info = plsc.get_sparse_core_info()
    nw = info.num_cores * info.num_subcores
    assert s % nw == 0, (s, nw)
    rows_per_w = s // nw
    ns = info.num_subcores
    ch = rows_per_w
    while ch * d * 4 > 256 * 1024:
        ch //= 2
    n_ch = rows_per_w // ch
    b_sp = b // 2

    mesh = plsc.VectorSubcoreMesh(core_axis_name="c", subcore_axis_name="s")

    @functools.partial(
        pl.kernel,
        mesh=mesh,
        out_type=jax.ShapeDtypeStruct((b, s, d), jnp.float32),
        scratch_types=[
            pltpu.VMEM((ch, d), jnp.float32),
            pltpu.VMEM_SHARED((ns, ch, d), jnp.float32),
            pltpu.SemaphoreType.DMA,
            pltpu.SemaphoreType.DMA,
            pltpu.SemaphoreType.DMA,
            pltpu.SemaphoreType.DMA,
        ],
    )
    def prog(emb_hbm, out_hbm, tbuf, spbuf, lsem, lsem2, ssem, ssem2):
        sid = lax.axis_index("s")
        wid = sid * info.num_cores + lax.axis_index("c")
        base0 = wid * rows_per_w
        for c in range(n_ch):
            base = base0 + c * ch
            src = emb_hbm.at[pl.ds(base, ch)]
            lt = pltpu.async_copy(src, tbuf, lsem)
            ls = pltpu.async_copy(src, spbuf.at[sid], lsem2)
            lt.wait()
            hs = [
                pltpu.async_copy(tbuf, out_hbm.at[bb, pl.ds(base, ch)], ssem)
                for bb in range(b - b_sp)
            ]
            ls.wait()
            hs += [
                pltpu.async_copy(
                    spbuf.at[sid], out_hbm.at[bb, pl.ds(base, ch)], ssem2
                )
                for bb in range(b - b_sp, b)
            ]
            for h in hs:
                h.wait()

    return prog


def kernel(x, emb_weight):
    b, s, _ = x.shape
    d = emb_weight.shape[1]
    if s >= MAX_LEN:
        raise NotImplementedError("s >= MAX_SEQ_LEN not exercised by this problem")
    prog = _sc_broadcast_rows(b, s, d)
    return prog(emb_weight)


# speedup vs baseline: 45.0064x; 1.1273x over previous
import functools

import jax
import jax.numpy as jnp
from jax import lax
from jax.experimental import pallas as pl
from jax.experimental.pallas import tpu as pltpu
from jax.experimental.pallas import tpu_sc as plsc

MAX_LEN = 8192


def _sc_broadcast_rows(b, s, d):
    info = plsc.get_sparse_core_info()
    nw = info.num_cores * info.num_subcores
    assert s % nw == 0, (s, nw)
    rows_per_w = s // nw
    chunks = [48, 48, 32] if rows_per_w == 128 else [rows_per_w]
    assert sum(chunks) == rows_per_w
    ch_max = max(chunks)
    nbuf = 2 if len(chunks) > 1 else 1
    offs = [sum(chunks[:i]) for i in range(len(chunks))]

    mesh = plsc.VectorSubcoreMesh(core_axis_name="c", subcore_axis_name="s")

    @functools.partial(
        pl.kernel,
        mesh=mesh,
        out_type=jax.ShapeDtypeStruct((b, s, d), jnp.float32),
        scratch_types=[
            pltpu.VMEM((nbuf, ch_max, d), jnp.float32),
            pltpu.SemaphoreType.DMA((nbuf,)),
            pltpu.SemaphoreType.DMA((nbuf,)),
        ],
    )
    def prog(emb_hbm, out_hbm, buf, lsem, ssem):
        wid = lax.axis_index("s") * info.num_cores + lax.axis_index("c")
        base0 = wid * rows_per_w

        def load(c, slot):
            n = chunks[c]
            return pltpu.async_copy(
                emb_hbm.at[pl.ds(base0 + offs[c], n)],
                buf.at[slot, pl.ds(0, n)],
                lsem.at[slot],
            )

        def stores(c, slot):
            n = chunks[c]
            return [
                pltpu.async_copy(
                    buf.at[slot, pl.ds(0, n)],
                    out_hbm.at[bb, pl.ds(base0 + offs[c], n)],
                    ssem.at[slot],
                )
                for bb in range(b)
            ]

        n_ch = len(chunks)
        load_h = {0: load(0, 0)}
        store_h = {}
        for c in range(n_ch):
            slot = c % nbuf
            load_h[c].wait()
            store_h[c] = stores(c, slot)
            nc = c + 1
            if nc < n_ch:
                if nc - nbuf >= 0:
                    for h in store_h[nc - nbuf]:
                        h.wait()
                load_h[nc] = load(nc, nc % nbuf)
        for c in range(max(0, n_ch - nbuf), n_ch):
            for h in store_h[c]:
                h.wait()

    return prog


def kernel(x, emb_weight):
    b, s, _ = x.shape
    d = emb_weight.shape[1]
    if s >= MAX_LEN:
        raise NotImplementedError("s >= MAX_SEQ_LEN not exercised by this problem")
    prog = _sc_broadcast_rows(b, s, d)
    return prog(emb_weight)
